# baseline (speedup 1.0000x reference)
import functools

import jax
import jax.numpy as jnp
from jax import lax
from jax.experimental import pallas as pl
from jax.experimental.pallas import tpu as pltpu


def _pipeline_kernel(a_hbm, p_hbm, n_hbm, o_ref, bufs, sems, *,
                     margin, rows_per_core, tile_rows, num_steps, nbuf):
    core = pl.program_id(0)
    base = core * rows_per_core
    srcs = (a_hbm, p_hbm, n_hbm)

    def start(s):
        slot = s % nbuf
        row0 = base + s * tile_rows
        for j in range(3):
            pltpu.make_async_copy(
                srcs[j].at[pl.ds(row0, tile_rows), :],
                bufs.at[j, slot], sems.at[j, slot]).start()

    def wait(s):
        slot = s % nbuf
        for j in range(3):
            pltpu.make_async_copy(
                bufs.at[j, slot], bufs.at[j, slot], sems.at[j, slot]).wait()

    for s in range(min(nbuf, num_steps)):
        start(s)

    total = None
    for s in range(num_steps):
        wait(s)
        slot = s % nbuf
        a = bufs[0, slot]
        p = bufs[1, slot]
        n = bufs[2, slot]
        dp = a - p
        dn = a - n
        diff = dp * dp - dn * dn
        d = jnp.sum(diff, axis=-1, keepdims=True)
        per_row = jnp.maximum(d + margin, 0.0)
        ts = jnp.sum(per_row)
        total = ts if total is None else total + ts
        nxt = s + nbuf
        if nxt < num_steps:
            start(nxt)

    o_ref[...] = jnp.broadcast_to(jnp.reshape(total, (1, 1)), o_ref.shape)


def kernel(anchor, positive, negative, margin=1.0, tile_rows=None,
           nbuf=4, interpret=False):
    assert anchor.shape == positive.shape == negative.shape
    feat = anchor.shape[-1]
    anchor = anchor.reshape(-1, feat)
    positive = positive.reshape(-1, feat)
    negative = negative.reshape(-1, feat)
    batch = anchor.shape[0]
    assert batch % 16 == 0

    rows_per_core = batch // 2
    if tile_rows is None:
        tile_rows = 4096
        while rows_per_core % tile_rows or (tile_rows > 8 and
                                            rows_per_core // tile_rows < 2):
            tile_rows //= 2
    num_steps = rows_per_core // tile_rows
    assert num_steps * tile_rows == rows_per_core

    kernel_fn = functools.partial(
        _pipeline_kernel, margin=float(margin), rows_per_core=rows_per_core,
        tile_rows=tile_rows, num_steps=num_steps, nbuf=nbuf)

    hbm_spec = pl.BlockSpec(memory_space=pl.ANY)

    partial = pl.pallas_call(
        kernel_fn,
        out_shape=jax.ShapeDtypeStruct((2 * 8, 128), jnp.float32),
        grid=(2,),
        in_specs=[hbm_spec, hbm_spec, hbm_spec],
        out_specs=pl.BlockSpec((8, 128), lambda c: (c, 0)),
        scratch_shapes=[
            pltpu.VMEM((3, nbuf, tile_rows, feat), jnp.float32),
            pltpu.SemaphoreType.DMA((3, nbuf)),
        ],
        compiler_params=pltpu.CompilerParams(
            dimension_semantics=("parallel",),
            vmem_limit_bytes=56 * 1024 * 1024),
        interpret=interpret,
    )(anchor, positive, negative)

    return jnp.sum(partial) / (batch * 8.0 * 128.0)


# speedup vs baseline: 1.1864x; 1.1864x over previous
import functools

import jax
import jax.numpy as jnp
from jax import lax
from jax.experimental import pallas as pl
from jax.experimental.pallas import tpu as pltpu


def _ceil_div(a, b):
    return -(-a // b)


def _loss_tile_kernel(a_ref, p_ref, n_ref, o_ref, *,
                      margin, rows_total, tile_rows, inner, need_mask):
    i = pl.program_id(1)

    @pl.when(i == 0)
    def _init():
        o_ref[...] = jnp.zeros_like(o_ref)

    a = a_ref[...].astype(jnp.float32)
    p = p_ref[...].astype(jnp.float32)
    n = n_ref[...].astype(jnp.float32)

    dp = a - p
    dn = a - n
    diff = dp * dp - dn * dn
    d = jnp.sum(diff, axis=-1, keepdims=True)
    per_row = jnp.maximum(d + margin, 0.0)

    if need_mask:
        tile = pl.program_id(0) * inner + i
        rows = tile * tile_rows + lax.broadcasted_iota(
            jnp.int32, per_row.shape, 0)
        per_row = jnp.where(rows < rows_total, per_row, 0.0)

    tile_sum = jnp.sum(per_row, axis=0, keepdims=True)
    o_ref[...] += jnp.broadcast_to(tile_sum, o_ref.shape)


def kernel(anchor, positive, negative, margin=1.0, tile_rows=None):
    assert anchor.shape == positive.shape == negative.shape
    feat = anchor.shape[-1]
    anchor = anchor.reshape(-1, feat)
    positive = positive.reshape(-1, feat)
    negative = negative.reshape(-1, feat)
    batch = anchor.shape[0]

    lane_cols = _ceil_div(feat, 128) * 128
    itemsize = jnp.dtype(anchor.dtype).itemsize
    if tile_rows is None:
        tile_rows = max(8, (2 * 1024 * 1024 // (lane_cols * itemsize))
                        // 8 * 8)
        if tile_rows >= batch:
            tile_rows = batch
    tile_rows = int(tile_rows)
    assert tile_rows == batch or tile_rows % 8 == 0

    num_tiles = _ceil_div(batch, tile_rows)
    outer = 2 if num_tiles >= 2 else 1
    inner = _ceil_div(num_tiles, outer)
    need_mask = (outer * inner * tile_rows != batch)

    if outer * inner == num_tiles:
        def row_block(o, i):
            return (o * inner + i, 0)
    else:
        def row_block(o, i):
            return (jnp.minimum(o * inner + i, num_tiles - 1), 0)

    kernel_fn = functools.partial(
        _loss_tile_kernel, margin=float(margin), rows_total=batch,
        tile_rows=tile_rows, inner=inner, need_mask=need_mask)

    in_spec = pl.BlockSpec((tile_rows, feat), row_block)

    partial = pl.pallas_call(
        kernel_fn,
        out_shape=jax.ShapeDtypeStruct((outer * 8, 128), jnp.float32),
        grid=(outer, inner),
        in_specs=[in_spec, in_spec, in_spec],
        out_specs=pl.BlockSpec((8, 128), lambda o, i: (o, 0)),
        compiler_params=pltpu.CompilerParams(
            dimension_semantics=("parallel", "arbitrary"),
            vmem_limit_bytes=48 * 1024 * 1024),
    )(anchor, positive, negative)

    return jnp.sum(partial) / (batch * 8.0 * 128.0)
